# baseline (speedup 1.0000x reference)
import functools
import math

import jax
import jax.numpy as jnp
from jax import lax
from jax.experimental import pallas as pl
from jax.experimental.pallas import tpu as pltpu
from jax.experimental.pallas import tpu_sc as plsc

D_MODEL = 64
LANES = 16
IDX_W = 128
GATHERS_PER_CHUNK = 4
CHUNK = IDX_W * GATHERS_PER_CHUNK


@functools.partial(jax.jit, static_argnames=("n",))
def _sc_embed(x2d, table, n):
    info = plsc.get_sparse_core_info()
    nw = info.num_cores * info.num_subcores
    per_w = n // nw
    n_chunks = per_w // CHUNK
    scale = math.sqrt(float(D_MODEL))
    mesh = plsc.VectorSubcoreMesh(core_axis_name="c", subcore_axis_name="s")

    @functools.partial(
        pl.kernel,
        mesh=mesh,
        out_type=jax.ShapeDtypeStruct((n, D_MODEL), jnp.float32),
        scratch_types=[
            pltpu.VMEM((GATHERS_PER_CHUNK, IDX_W), jnp.int32),
            pltpu.VMEM((CHUNK, D_MODEL), jnp.float32),
            pltpu.SemaphoreType.DMA,
        ],
        compiler_params=pltpu.CompilerParams(use_tc_tiling_on_sc=False),
    )
    def k(x_hbm, tab_hbm, out_hbm, idx_v, rows_v, sem):
        wid = lax.axis_index("s") * info.num_cores + lax.axis_index("c")
        row0 = wid * (per_w // IDX_W)
        base = wid * per_w

        def chunk_body(c, _):
            pltpu.sync_copy(
                x_hbm.at[pl.ds(row0 + c * GATHERS_PER_CHUNK, GATHERS_PER_CHUNK)],
                idx_v)
            for g in range(GATHERS_PER_CHUNK):
                pltpu.async_copy(
                    tab_hbm.at[idx_v.at[g]],
                    rows_v.at[pl.ds(g * IDX_W, IDX_W)],
                    sem)
            for g in range(GATHERS_PER_CHUNK):
                pltpu.make_async_copy(
                    tab_hbm.at[idx_v.at[g]],
                    rows_v.at[pl.ds(g * IDX_W, IDX_W)],
                    sem).wait()

            def scale_row(i, _):
                for j in range(D_MODEL // LANES):
                    sl = (i, pl.ds(j * LANES, LANES))
                    rows_v[sl] = rows_v[sl] * scale
                return 0

            lax.fori_loop(0, CHUNK, scale_row, 0, unroll=2)
            pltpu.sync_copy(rows_v, out_hbm.at[pl.ds(base + c * CHUNK, CHUNK)])
            return 0

        lax.fori_loop(0, n_chunks, chunk_body, 0)

    return k(x2d, table)


def kernel(x, table):
    b, l = x.shape
    n = b * l
    out = _sc_embed(x.reshape(n // IDX_W, IDX_W), table, n)
    return out.reshape(b, l, D_MODEL)


# speedup vs baseline: 1.2134x; 1.2134x over previous
import functools
import math

import jax
import jax.numpy as jnp
from jax import lax
from jax.experimental import pallas as pl
from jax.experimental.pallas import tpu as pltpu
from jax.experimental.pallas import tpu_sc as plsc

D_MODEL = 64
D_PAD = 128
LANES = 16
IDX_W = 128
GATHERS_PER_CHUNK = 4
CHUNK = IDX_W * GATHERS_PER_CHUNK


@functools.partial(jax.jit, static_argnames=("n",))
def _sc_embed(x2d, tpad, n):
    info = plsc.get_sparse_core_info()
    nw = info.num_cores * info.num_subcores
    per_w = n // nw
    n_chunks = per_w // CHUNK
    scale = math.sqrt(float(D_MODEL))
    mesh = plsc.VectorSubcoreMesh(core_axis_name="c", subcore_axis_name="s")

    @functools.partial(
        pl.kernel,
        mesh=mesh,
        out_type=jax.ShapeDtypeStruct((n, D_PAD), jnp.float32),
        scratch_types=[
            pltpu.VMEM((GATHERS_PER_CHUNK, IDX_W), jnp.int32),
            pltpu.VMEM((CHUNK, D_PAD), jnp.float32),
            pltpu.SemaphoreType.DMA,
        ],
        compiler_params=pltpu.CompilerParams(use_tc_tiling_on_sc=True),
    )
    def k(x_hbm, tab_hbm, out_hbm, idx_v, rows_v, sem):
        wid = lax.axis_index("s") * info.num_cores + lax.axis_index("c")
        row0 = wid * (per_w // IDX_W)
        base = wid * per_w

        def chunk_body(c, _):
            pltpu.sync_copy(
                x_hbm.at[pl.ds(row0 + c * GATHERS_PER_CHUNK, GATHERS_PER_CHUNK)],
                idx_v)
            for g in range(GATHERS_PER_CHUNK):
                pltpu.async_copy(
                    tab_hbm.at[idx_v.at[g]],
                    rows_v.at[pl.ds(g * IDX_W, IDX_W)],
                    sem)
            for g in range(GATHERS_PER_CHUNK):
                pltpu.make_async_copy(
                    tab_hbm.at[idx_v.at[g]],
                    rows_v.at[pl.ds(g * IDX_W, IDX_W)],
                    sem).wait()

            def scale_row(i, _):
                for j in range(D_MODEL // LANES):
                    sl = (i, pl.ds(j * LANES, LANES))
                    rows_v[sl] = rows_v[sl] * scale
                return 0

            lax.fori_loop(0, CHUNK, scale_row, 0, unroll=2)
            pltpu.sync_copy(rows_v, out_hbm.at[pl.ds(base + c * CHUNK, CHUNK)])
            return 0

        lax.fori_loop(0, n_chunks, chunk_body, 0)

    return k(x2d, tpad)


def kernel(x, table):
    b, l = x.shape
    n = b * l
    tpad = jnp.pad(table, ((0, 0), (0, D_PAD - D_MODEL)))
    out = _sc_embed(x.reshape(n // IDX_W, IDX_W), tpad, n)
    return out[:, :D_MODEL].reshape(b, l, D_MODEL)
